# Initial kernel scaffold; baseline (speedup 1.0000x reference)
import jax
import jax.numpy as jnp
from jax.experimental import pallas as pl


def kernel(x, scale):
    raise NotImplementedError("write your pallas kernel here")


# speedup vs baseline: 28.7975x; 28.7975x over previous
import functools

import jax
import jax.numpy as jnp
from jax import lax
from jax.experimental import pallas as pl
from jax.experimental.pallas import tpu as pltpu
from jax.experimental.pallas import tpu_sc as plsc

_GAMMA = 0.999
_P = 0.9995

_L = 16
_NC = 2
_NS = 16
_NW = _NC * _NS
_NBINS = 1 << 16
_CHUNK = 8192
_NVEC = _CHUNK // _L


def _keys_of(v):
    v = jnp.where(v < _GAMMA, v, jnp.float32(_GAMMA))
    t = lax.bitcast_convert_type(v, jnp.int32)
    m = lax.shift_right_arithmetic(t, 31)
    return lax.bitwise_xor(t, lax.bitwise_or(m, jnp.int32(-2147483648)))


def _zero_hist(hist):
    zeros = jnp.zeros((_L,), jnp.int32)

    def z(i, c):
        hist[pl.ds(i * _L, _L)] = zeros
        return c

    lax.fori_loop(0, _NBINS // _L, z, 0)


def _hist_hi_body(total):
    per_w = total // _NW
    nchunk = per_w // _CHUNK

    def body(x_hbm, out_hbm, buf, hist, sem):
        wid = lax.axis_index("s") * _NC + lax.axis_index("c")
        base = wid * per_w
        _zero_hist(hist)
        ones = jnp.ones((_L,), jnp.int32)

        def chunk(g, c):
            start = pl.multiple_of(base + g * _CHUNK, _CHUNK)
            pltpu.async_copy(x_hbm.at[pl.ds(start, _CHUNK)], buf, sem).wait()

            def vec(i, c2):
                key = _keys_of(buf[pl.ds(i * _L, _L)])
                hi = lax.shift_right_logical(key, 16)
                plsc.addupdate_scatter(hist, [hi], ones)
                return c2

            lax.fori_loop(0, _NVEC, vec, 0)
            return c

        lax.fori_loop(0, nchunk, chunk, 0)
        pltpu.sync_copy(hist, out_hbm.at[wid])

    return body


def _hist_lo_body(total):
    per_w = total // _NW
    nchunk = per_w // _CHUNK

    def body(x_hbm, filt_hbm, out_hbm, buf, filtv, hist, sem):
        wid = lax.axis_index("s") * _NC + lax.axis_index("c")
        base = wid * per_w
        _zero_hist(hist)
        pltpu.sync_copy(filt_hbm, filtv)
        filts = filtv[...]
        ones = jnp.ones((_L,), jnp.int32)

        def chunk(g, c):
            start = pl.multiple_of(base + g * _CHUNK, _CHUNK)
            pltpu.async_copy(x_hbm.at[pl.ds(start, _CHUNK)], buf, sem).wait()

            def vec(i, c2):
                key = _keys_of(buf[pl.ds(i * _L, _L)])
                hi = lax.shift_right_logical(key, 16)
                lo = lax.bitwise_and(key, jnp.int32(0xFFFF))
                plsc.addupdate_scatter(hist, [lo], ones, mask=hi == filts)
                return c2

            lax.fori_loop(0, _NVEC, vec, 0)
            return c

        lax.fori_loop(0, nchunk, chunk, 0)
        pltpu.sync_copy(hist, out_hbm.at[wid])

    return body


@functools.cache
def _make_sc_kernels(total):
    mesh = plsc.VectorSubcoreMesh(core_axis_name="c", subcore_axis_name="s")
    out_type = jax.ShapeDtypeStruct((_NW, _NBINS), jnp.int32)
    params = pltpu.CompilerParams(needs_layout_passes=False)
    hist_hi = pl.kernel(
        _hist_hi_body(total),
        out_type=out_type,
        mesh=mesh,
        compiler_params=params,
        scratch_types=[
            pltpu.VMEM((_CHUNK,), jnp.float32),
            pltpu.VMEM((_NBINS,), jnp.int32),
            pltpu.SemaphoreType.DMA,
        ],
    )
    hist_lo = pl.kernel(
        _hist_lo_body(total),
        out_type=out_type,
        mesh=mesh,
        compiler_params=params,
        scratch_types=[
            pltpu.VMEM((_CHUNK,), jnp.float32),
            pltpu.VMEM((_L,), jnp.int32),
            pltpu.VMEM((_NBINS,), jnp.int32),
            pltpu.SemaphoreType.DMA,
        ],
    )
    return hist_hi, hist_lo


def _clamp_body(x_ref, o_ref):
    v = x_ref[...]
    o_ref[...] = jnp.where(v < _GAMMA, v, jnp.float32(_GAMMA))


def _clamp2d(x2d):
    rows, cols = x2d.shape
    br = 512
    return pl.pallas_call(
        _clamp_body,
        grid=(rows // br,),
        in_specs=[pl.BlockSpec((br, cols), lambda i: (i, 0))],
        out_specs=pl.BlockSpec((br, cols), lambda i: (i, 0)),
        out_shape=jax.ShapeDtypeStruct((rows, cols), jnp.float32),
    )(x2d)


def kernel(x, scale):
    shp = x.shape
    total = 1
    for s in shp:
        total *= s
    xflat = x.reshape(total)

    out = _clamp2d(x.reshape(total // shp[-1], shp[-1])).reshape(shp)

    hist_hi, hist_lo = _make_sc_kernels(total)
    k_rank = int(total * _P) - 1

    h1 = hist_hi(xflat).sum(axis=0)
    c1 = jnp.cumsum(h1)
    b = jnp.searchsorted(c1, k_rank + 1, side="left").astype(jnp.int32)
    r = k_rank - (c1[b] - h1[b])

    filt = jnp.full((_L,), b, jnp.int32)
    h2 = hist_lo(xflat, filt).sum(axis=0)
    c2 = jnp.cumsum(h2)
    lo = jnp.searchsorted(c2, r + 1, side="left").astype(jnp.int32)

    key_u = (b.astype(jnp.uint32) << 16) | lo.astype(jnp.uint32)
    top = jnp.uint32(0x80000000)
    u = jnp.where(key_u >= top, key_u ^ top, ~key_u)
    val = lax.bitcast_convert_type(u, jnp.float32)
    new_scale = jnp.maximum(val, scale)
    return out, new_scale


# speedup vs baseline: 98.7609x; 3.4295x over previous
import functools
import struct

import jax
import jax.numpy as jnp
from jax import lax
from jax.experimental import pallas as pl
from jax.experimental.pallas import tpu as pltpu
from jax.experimental.pallas import tpu_sc as plsc

_GAMMA = 0.999
_P = 0.9995

_L = 16
_NC = 2
_NS = 16
_NW = _NC * _NS
_NBINS = 1 << 16
_CHUNK = 16384
_UNROLL = 8

_GBITS = struct.unpack("<i", struct.pack("<f", _GAMMA))[0]
_KEY_G = (_GBITS | 0x80000000) & 0xFFFFFFFF
_HI_G = _KEY_G >> 16
_LO_G = _KEY_G & 0xFFFF


def _zero_hist(hist):
    zeros = jnp.zeros((_L,), jnp.int32)

    @plsc.parallel_loop(0, _NBINS // _L, 1, unroll=8)
    def _(i):
        hist[pl.ds(i * _L, _L)] = zeros


def _hist_body(total, lo_pass):
    per_w = total // _NW
    nchunk = per_w // _CHUNK
    nouter = nchunk // 2

    def process(buf, hist, filts, acc):
        ones = jnp.ones((_L,), jnp.int32)
        one = jnp.int32(1)
        zero = jnp.int32(0)
        gam = jnp.float32(_GAMMA)
        msb = jnp.int32(-2147483648)

        def vec(i, acc):
            v = buf[pl.ds(i * _L, _L)]
            m = v < gam
            cl = jnp.where(m, v, gam)
            t = lax.bitcast_convert_type(cl, jnp.int32)
            s = lax.shift_right_arithmetic(t, 31)
            key = lax.bitwise_xor(t, lax.bitwise_or(s, msb))
            hi = lax.shift_right_logical(key, 16)
            if lo_pass:
                lo = lax.bitwise_and(key, jnp.int32(0xFFFF))
                plsc.addupdate_scatter(
                    hist, [lo], ones, mask=jnp.logical_and(m, hi == filts)
                )
            else:
                plsc.addupdate_scatter(hist, [hi], ones, mask=m)
            return acc + jnp.where(m, zero, one)

        return plsc.parallel_loop(0, _CHUNK // _L, 1, unroll=_UNROLL, carry=acc)(
            vec
        )

    def body(x_hbm, filt_hbm, out_hbm, buf0, buf1, filtv, hist, sem0, sem1):
        wid = lax.axis_index("s") * _NC + lax.axis_index("c")
        base = wid * per_w
        _zero_hist(hist)
        if lo_pass:
            pltpu.sync_copy(filt_hbm, filtv)
            filts = filtv[...]
        else:
            filts = None

        pltpu.async_copy(x_hbm.at[pl.ds(base, _CHUNK)], buf0, sem0)

        def outer(h, acc):
            s1 = pl.multiple_of(base + (2 * h + 1) * _CHUNK, _CHUNK)
            pltpu.async_copy(x_hbm.at[pl.ds(s1, _CHUNK)], buf1, sem1)
            pltpu.make_async_copy(x_hbm.at[pl.ds(0, _CHUNK)], buf0, sem0).wait()
            acc = process(buf0, hist, filts, acc)

            @pl.when(h < nouter - 1)
            def _():
                s0 = pl.multiple_of(base + (2 * h + 2) * _CHUNK, _CHUNK)
                pltpu.async_copy(x_hbm.at[pl.ds(s0, _CHUNK)], buf0, sem0)

            pltpu.make_async_copy(x_hbm.at[pl.ds(0, _CHUNK)], buf1, sem1).wait()
            acc = process(buf1, hist, filts, acc)
            return acc

        acc = lax.fori_loop(0, nouter, outer, jnp.zeros((_L,), jnp.int32))
        total_g = jnp.sum(acc)
        lane0 = lax.iota(jnp.int32, _L) == 0
        gvec = jnp.full((_L,), total_g, jnp.int32)
        if lo_pass:
            gidx = jnp.full((_L,), _LO_G, jnp.int32)
            gmask = jnp.logical_and(lane0, filts == _HI_G)
        else:
            gidx = jnp.full((_L,), _HI_G, jnp.int32)
            gmask = lane0
        plsc.addupdate_scatter(hist, [gidx], gvec, mask=gmask)
        pltpu.sync_copy(hist, out_hbm.at[wid])

    if lo_pass:
        return body

    def body_hi(x_hbm, out_hbm, buf0, buf1, hist, sem0, sem1):
        return body(x_hbm, None, out_hbm, buf0, buf1, None, hist, sem0, sem1)

    return body_hi


@functools.cache
def _make_sc_kernels(total):
    mesh = plsc.VectorSubcoreMesh(core_axis_name="c", subcore_axis_name="s")
    out_type = jax.ShapeDtypeStruct((_NW, _NBINS), jnp.int32)
    params = pltpu.CompilerParams(needs_layout_passes=False)
    hist_hi = pl.kernel(
        _hist_body(total, lo_pass=False),
        out_type=out_type,
        mesh=mesh,
        compiler_params=params,
        scratch_types=[
            pltpu.VMEM((_CHUNK,), jnp.float32),
            pltpu.VMEM((_CHUNK,), jnp.float32),
            pltpu.VMEM((_NBINS,), jnp.int32),
            pltpu.SemaphoreType.DMA,
            pltpu.SemaphoreType.DMA,
        ],
    )
    hist_lo = pl.kernel(
        _hist_body(total, lo_pass=True),
        out_type=out_type,
        mesh=mesh,
        compiler_params=params,
        scratch_types=[
            pltpu.VMEM((_CHUNK,), jnp.float32),
            pltpu.VMEM((_CHUNK,), jnp.float32),
            pltpu.VMEM((_L,), jnp.int32),
            pltpu.VMEM((_NBINS,), jnp.int32),
            pltpu.SemaphoreType.DMA,
            pltpu.SemaphoreType.DMA,
        ],
    )
    return hist_hi, hist_lo


def _clamp_body(x_ref, o_ref):
    v = x_ref[...]
    o_ref[...] = jnp.where(v < _GAMMA, v, jnp.float32(_GAMMA))


def _clamp2d(x2d):
    rows, cols = x2d.shape
    br = 512
    return pl.pallas_call(
        _clamp_body,
        grid=(rows // br,),
        in_specs=[pl.BlockSpec((br, cols), lambda i: (i, 0))],
        out_specs=pl.BlockSpec((br, cols), lambda i: (i, 0)),
        out_shape=jax.ShapeDtypeStruct((rows, cols), jnp.float32),
    )(x2d)


def kernel(x, scale):
    shp = x.shape
    total = 1
    for s in shp:
        total *= s
    xflat = x.reshape(total)

    out = _clamp2d(x.reshape(total // shp[-1], shp[-1])).reshape(shp)

    hist_hi, hist_lo = _make_sc_kernels(total)
    k_rank = int(total * _P) - 1

    h1 = hist_hi(xflat).sum(axis=0)
    c1 = jnp.cumsum(h1)
    b = jnp.searchsorted(c1, k_rank + 1, side="left").astype(jnp.int32)
    r = k_rank - (c1[b] - h1[b])

    filt = jnp.full((_L,), b, jnp.int32)
    h2 = hist_lo(xflat, filt).sum(axis=0)
    c2 = jnp.cumsum(h2)
    lo = jnp.searchsorted(c2, r + 1, side="left").astype(jnp.int32)

    key_u = (b.astype(jnp.uint32) << 16) | lo.astype(jnp.uint32)
    top = jnp.uint32(0x80000000)
    u = jnp.where(key_u >= top, key_u ^ top, ~key_u)
    val = lax.bitcast_convert_type(u, jnp.float32)
    new_scale = jnp.maximum(val, scale)
    return out, new_scale


# speedup vs baseline: 126.6183x; 1.2821x over previous
import functools
import struct

import jax
import jax.numpy as jnp
from jax import lax
from jax.experimental import pallas as pl
from jax.experimental.pallas import tpu as pltpu
from jax.experimental.pallas import tpu_sc as plsc

_GAMMA = 0.999
_P = 0.9995

_L = 16
_NC = 2
_NS = 16
_NW = _NC * _NS
_NBINS = 1 << 16
_CROWS = 8
_UNROLL = 2

_GBITS = struct.unpack("<i", struct.pack("<f", _GAMMA))[0]
_KEY_G = (_GBITS | 0x80000000) & 0xFFFFFFFF
_HI_G = _KEY_G >> 16
_LO_G = _KEY_G & 0xFFFF


def _zero_hist(hist):
    zeros = jnp.zeros((_L,), jnp.int32)

    @plsc.parallel_loop(0, _NBINS // _L, 1, unroll=8)
    def _(i):
        hist[pl.ds(i * _L, _L)] = zeros


def _hist_body(rows, cols, lo_pass):
    rows_w = rows // _NW
    nchunk = rows_w // _CROWS
    nouter = nchunk // 2
    vec_per_row = cols // _L

    def process(buf, hist, filts, acc):
        ones = jnp.ones((_L,), jnp.int32)
        one = jnp.int32(1)
        zero = jnp.int32(0)
        gam = jnp.float32(_GAMMA)
        msb = jnp.int32(-2147483648)

        def vec(i, acc):
            for r in range(_CROWS):
                v = buf[r, pl.ds(i * _L, _L)]
                m = v < gam
                t = lax.bitcast_convert_type(v, jnp.int32)
                s = lax.shift_right_arithmetic(t, 31)
                key = lax.bitwise_xor(t, lax.bitwise_or(s, msb))
                hi = lax.shift_right_logical(key, 16)
                if lo_pass:
                    lo = lax.bitwise_and(key, jnp.int32(0xFFFF))
                    plsc.addupdate_scatter(
                        hist, [lo], ones, mask=jnp.logical_and(m, hi == filts)
                    )
                else:
                    plsc.addupdate_scatter(hist, [hi], ones, mask=m)
                    acc = acc + jnp.where(m, zero, one)
            return acc

        return plsc.parallel_loop(0, vec_per_row, 1, unroll=_UNROLL, carry=acc)(
            vec
        )

    def body(x_hbm, filt_hbm, out_hbm, gout_hbm, buf0, buf1, filtv, hist, sem0,
             sem1):
        wid = lax.axis_index("s") * _NC + lax.axis_index("c")
        base = wid * rows_w
        _zero_hist(hist)
        if lo_pass:
            pltpu.sync_copy(filt_hbm, filtv)
            filts = filtv[...]
        else:
            filts = None

        pltpu.async_copy(x_hbm.at[pl.ds(base, _CROWS)], buf0, sem0)

        def outer(h, acc):
            s1 = pl.multiple_of(base + (2 * h + 1) * _CROWS, _CROWS)
            pltpu.async_copy(x_hbm.at[pl.ds(s1, _CROWS)], buf1, sem1)
            pltpu.make_async_copy(x_hbm.at[pl.ds(0, _CROWS)], buf0, sem0).wait()
            acc = process(buf0, hist, filts, acc)

            @pl.when(h < nouter - 1)
            def _():
                s0 = pl.multiple_of(base + (2 * h + 2) * _CROWS, _CROWS)
                pltpu.async_copy(x_hbm.at[pl.ds(s0, _CROWS)], buf0, sem0)

            pltpu.make_async_copy(x_hbm.at[pl.ds(0, _CROWS)], buf1, sem1).wait()
            acc = process(buf1, hist, filts, acc)
            return acc

        acc = lax.fori_loop(0, nouter, outer, jnp.zeros((_L,), jnp.int32))
        if not lo_pass:
            total_g = jnp.sum(acc)
            lane0 = lax.iota(jnp.int32, _L) == 0
            gvec = jnp.full((_L,), total_g, jnp.int32)
            gidx = jnp.full((_L,), _HI_G, jnp.int32)
            plsc.addupdate_scatter(hist, [gidx], gvec, mask=lane0)
            gsel = jnp.where(lane0, gvec, jnp.zeros((_L,), jnp.int32))
            filtv[...] = gsel
            pltpu.sync_copy(filtv, gout_hbm.at[wid])
        pltpu.sync_copy(hist, out_hbm.at[wid])

    if lo_pass:
        def body_lo(x_hbm, filt_hbm, out_hbm, buf0, buf1, filtv, hist, sem0,
                    sem1):
            return body(x_hbm, filt_hbm, out_hbm, None, buf0, buf1, filtv,
                        hist, sem0, sem1)

        return body_lo

    def body_hi(x_hbm, out_hbm, gout_hbm, buf0, buf1, gv, hist, sem0, sem1):
        return body(x_hbm, None, out_hbm, gout_hbm, buf0, buf1, gv, hist,
                    sem0, sem1)

    return body_hi


@functools.cache
def _make_sc_kernels(rows, cols):
    mesh = plsc.VectorSubcoreMesh(core_axis_name="c", subcore_axis_name="s")
    hist_type = jax.ShapeDtypeStruct((_NW, _NBINS), jnp.int32)
    params = pltpu.CompilerParams(needs_layout_passes=False)
    hist_hi = pl.kernel(
        _hist_body(rows, cols, lo_pass=False),
        out_type=(hist_type, jax.ShapeDtypeStruct((_NW, _L), jnp.int32)),
        mesh=mesh,
        compiler_params=params,
        scratch_types=[
            pltpu.VMEM((_CROWS, cols), jnp.float32),
            pltpu.VMEM((_CROWS, cols), jnp.float32),
            pltpu.VMEM((_L,), jnp.int32),
            pltpu.VMEM((_NBINS,), jnp.int32),
            pltpu.SemaphoreType.DMA,
            pltpu.SemaphoreType.DMA,
        ],
    )
    hist_lo = pl.kernel(
        _hist_body(rows, cols, lo_pass=True),
        out_type=hist_type,
        mesh=mesh,
        compiler_params=params,
        scratch_types=[
            pltpu.VMEM((_CROWS, cols), jnp.float32),
            pltpu.VMEM((_CROWS, cols), jnp.float32),
            pltpu.VMEM((_L,), jnp.int32),
            pltpu.VMEM((_NBINS,), jnp.int32),
            pltpu.SemaphoreType.DMA,
            pltpu.SemaphoreType.DMA,
        ],
    )
    return hist_hi, hist_lo


def _clamp_body(x_ref, o_ref):
    v = x_ref[...]
    o_ref[...] = jnp.where(v < _GAMMA, v, jnp.float32(_GAMMA))


def _clamp2d(x2d):
    rows, cols = x2d.shape
    br = 512
    return pl.pallas_call(
        _clamp_body,
        grid=(rows // br,),
        in_specs=[pl.BlockSpec((br, cols), lambda i: (i, 0))],
        out_specs=pl.BlockSpec((br, cols), lambda i: (i, 0)),
        out_shape=jax.ShapeDtypeStruct((rows, cols), jnp.float32),
    )(x2d)


def kernel(x, scale):
    shp = x.shape
    total = 1
    for s in shp:
        total *= s
    cols = shp[-1]
    rows = total // cols
    x2d = x.reshape(rows, cols)

    out = _clamp2d(x2d).reshape(shp)

    hist_hi, hist_lo = _make_sc_kernels(rows, cols)
    k_rank = int(total * _P) - 1

    h1_rows, g_rows = hist_hi(x2d)
    h1 = h1_rows.sum(axis=0)
    gamma_total = g_rows.sum()
    c1 = jnp.cumsum(h1)
    b = jnp.searchsorted(c1, k_rank + 1, side="left").astype(jnp.int32)
    r = k_rank - (c1[b] - h1[b])

    filt = jnp.full((_L,), b, jnp.int32)
    h2 = hist_lo(x2d, filt).sum(axis=0)
    h2 = h2.at[_LO_G].add(jnp.where(b == _HI_G, gamma_total, 0))
    c2 = jnp.cumsum(h2)
    lo = jnp.searchsorted(c2, r + 1, side="left").astype(jnp.int32)

    key_u = (b.astype(jnp.uint32) << 16) | lo.astype(jnp.uint32)
    top = jnp.uint32(0x80000000)
    u = jnp.where(key_u >= top, key_u ^ top, ~key_u)
    val = lax.bitcast_convert_type(u, jnp.float32)
    new_scale = jnp.maximum(val, scale)
    return out, new_scale


# speedup vs baseline: 173.1275x; 1.3673x over previous
import functools
import struct

import jax
import jax.numpy as jnp
from jax import lax
from jax.experimental import pallas as pl
from jax.experimental.pallas import tpu as pltpu
from jax.experimental.pallas import tpu_sc as plsc

_GAMMA = 0.999
_P = 0.9995

_L = 16
_NC = 2
_NS = 16
_NW = _NC * _NS
_NBINS = 1 << 16
_CROWS = 8
_UNROLL = 2

_GBITS = struct.unpack("<i", struct.pack("<f", _GAMMA))[0]
_KEY_G = (_GBITS | 0x80000000) & 0xFFFFFFFF
_HI_G = _KEY_G >> 16
_LO_G = _KEY_G & 0xFFFF
_INF_BITS = 0x7F800000


def _zero_hist(hist):
    zeros = jnp.zeros((_L,), jnp.int32)

    @plsc.parallel_loop(0, _NBINS // _L, 1, unroll=8)
    def _(i):
        hist[pl.ds(i * _L, _L)] = zeros


def _hist_body(rows, cols, lo_pass):
    rows_w = rows // _NW
    nchunk = rows_w // _CROWS
    nouter = nchunk // 2
    vec_per_row = cols // _L

    def process(buf, hist, params):
        ones = jnp.ones((_L,), jnp.int32)
        gam = jnp.float32(_GAMMA)
        msb = jnp.int32(-2147483648)
        low16 = jnp.int32(0xFFFF)

        def vec(i):
            for r in range(_CROWS):
                v = buf[r, pl.ds(i * _L, _L)]
                t = lax.bitcast_convert_type(v, jnp.int32)
                if lo_pass:
                    bbv, xmv, thrv = params
                    m = v < thrv
                    em = lax.shift_right_logical(t, 16) == bbv
                    lo = lax.bitwise_and(lax.bitwise_xor(t, xmv), low16)
                    plsc.addupdate_scatter(
                        hist, [lo], ones, mask=jnp.logical_and(m, em)
                    )
                else:
                    m = v < gam
                    s = lax.shift_right_arithmetic(t, 31)
                    key = lax.bitwise_xor(t, lax.bitwise_or(s, msb))
                    hi = lax.shift_right_logical(key, 16)
                    plsc.addupdate_scatter(hist, [hi], ones, mask=m)

        plsc.parallel_loop(0, vec_per_row, 1, unroll=_UNROLL)(vec)

    def body(x_hbm, out_hbm, buf0, buf1, hist, sem0, sem1, params=None):
        wid = lax.axis_index("s") * _NC + lax.axis_index("c")
        base = wid * rows_w
        _zero_hist(hist)

        pltpu.async_copy(x_hbm.at[pl.ds(base, _CROWS)], buf0, sem0)

        def outer(h, c):
            s1 = pl.multiple_of(base + (2 * h + 1) * _CROWS, _CROWS)
            pltpu.async_copy(x_hbm.at[pl.ds(s1, _CROWS)], buf1, sem1)
            pltpu.make_async_copy(x_hbm.at[pl.ds(0, _CROWS)], buf0, sem0).wait()
            process(buf0, hist, params)

            @pl.when(h < nouter - 1)
            def _():
                s0 = pl.multiple_of(base + (2 * h + 2) * _CROWS, _CROWS)
                pltpu.async_copy(x_hbm.at[pl.ds(s0, _CROWS)], buf0, sem0)

            pltpu.make_async_copy(x_hbm.at[pl.ds(0, _CROWS)], buf1, sem1).wait()
            process(buf1, hist, params)
            return c

        lax.fori_loop(0, nouter, outer, 0)
        pltpu.sync_copy(hist, out_hbm.at[wid])

    if lo_pass:
        def body_lo(x_hbm, filt_hbm, out_hbm, buf0, buf1, filtv, hist, sem0,
                    sem1):
            pltpu.sync_copy(filt_hbm, filtv)
            bbv = filtv[0, :]
            xmv = filtv[1, :]
            thrv = lax.bitcast_convert_type(filtv[2, :], jnp.float32)
            body(x_hbm, out_hbm, buf0, buf1, hist, sem0, sem1,
                 params=(bbv, xmv, thrv))

        return body_lo

    def body_hi(x_hbm, out_hbm, buf0, buf1, hist, sem0, sem1):
        body(x_hbm, out_hbm, buf0, buf1, hist, sem0, sem1)

    return body_hi


@functools.cache
def _make_sc_kernels(rows, cols):
    mesh = plsc.VectorSubcoreMesh(core_axis_name="c", subcore_axis_name="s")
    hist_type = jax.ShapeDtypeStruct((_NW, _NBINS), jnp.int32)
    params = pltpu.CompilerParams(needs_layout_passes=False)
    hist_hi = pl.kernel(
        _hist_body(rows, cols, lo_pass=False),
        out_type=hist_type,
        mesh=mesh,
        compiler_params=params,
        scratch_types=[
            pltpu.VMEM((_CROWS, cols), jnp.float32),
            pltpu.VMEM((_CROWS, cols), jnp.float32),
            pltpu.VMEM((_NBINS,), jnp.int32),
            pltpu.SemaphoreType.DMA,
            pltpu.SemaphoreType.DMA,
        ],
    )
    hist_lo = pl.kernel(
        _hist_body(rows, cols, lo_pass=True),
        out_type=hist_type,
        mesh=mesh,
        compiler_params=params,
        scratch_types=[
            pltpu.VMEM((_CROWS, cols), jnp.float32),
            pltpu.VMEM((_CROWS, cols), jnp.float32),
            pltpu.VMEM((3, _L), jnp.int32),
            pltpu.VMEM((_NBINS,), jnp.int32),
            pltpu.SemaphoreType.DMA,
            pltpu.SemaphoreType.DMA,
        ],
    )
    return hist_hi, hist_lo


def _clamp_body(x_ref, o_ref):
    v = x_ref[...]
    o_ref[...] = jnp.where(v < _GAMMA, v, jnp.float32(_GAMMA))


def _clamp2d(x2d):
    rows, cols = x2d.shape
    br = 512
    return pl.pallas_call(
        _clamp_body,
        grid=(rows // br,),
        in_specs=[pl.BlockSpec((br, cols), lambda i: (i, 0))],
        out_specs=pl.BlockSpec((br, cols), lambda i: (i, 0)),
        out_shape=jax.ShapeDtypeStruct((rows, cols), jnp.float32),
    )(x2d)


def kernel(x, scale):
    shp = x.shape
    total = 1
    for s in shp:
        total *= s
    cols = shp[-1]
    rows = total // cols
    x2d = x.reshape(rows, cols)

    out = _clamp2d(x2d).reshape(shp)

    hist_hi, hist_lo = _make_sc_kernels(rows, cols)
    k_rank = int(total * _P) - 1
    kp1 = jnp.int32(k_rank + 1)

    h1 = hist_hi(x2d).sum(axis=0)
    c1 = jnp.cumsum(h1)
    gamma_total = jnp.int32(total) - c1[_NBINS - 1]
    bins = lax.iota(jnp.int32, _NBINS)
    c1f = c1 + jnp.where(bins >= _HI_G, gamma_total, 0)
    b = jnp.sum((c1f < kp1).astype(jnp.int32))
    h1b = h1[b] + jnp.where(b == _HI_G, gamma_total, 0)
    r = k_rank - (c1f[b] - h1b)

    pos = b >= jnp.int32(0x8000)
    bb = jnp.where(pos, b ^ jnp.int32(0x8000), (~b) & jnp.int32(0xFFFF))
    xm = jnp.where(pos, jnp.int32(0), jnp.int32(0xFFFF))
    thr = jnp.where(b == _HI_G, jnp.int32(_GBITS), jnp.int32(_INF_BITS))
    filt = jnp.stack([
        jnp.full((_L,), bb, jnp.int32),
        jnp.full((_L,), xm, jnp.int32),
        jnp.full((_L,), thr, jnp.int32),
    ])

    h2 = hist_lo(x2d, filt).sum(axis=0)
    h2 = h2.at[_LO_G].add(jnp.where(b == _HI_G, gamma_total, 0))
    c2 = jnp.cumsum(h2)
    rp1 = (r + 1).astype(jnp.int32)
    lo = jnp.sum((c2 < rp1).astype(jnp.int32))

    key_u = (b.astype(jnp.uint32) << 16) | lo.astype(jnp.uint32)
    top = jnp.uint32(0x80000000)
    u = jnp.where(key_u >= top, key_u ^ top, ~key_u)
    val = lax.bitcast_convert_type(u, jnp.float32)
    new_scale = jnp.maximum(val, scale)
    return out, new_scale


# speedup vs baseline: 174.8563x; 1.0100x over previous
import functools
import struct

import jax
import jax.numpy as jnp
from jax import lax
from jax.experimental import pallas as pl
from jax.experimental.pallas import tpu as pltpu
from jax.experimental.pallas import tpu_sc as plsc

_GAMMA = 0.999
_P = 0.9995

_L = 16
_NC = 2
_NS = 16
_NW = _NC * _NS
_NBINS = 1 << 16
_HALF = 1 << 15
_CROWS = 8
_UNROLL = 2

_GBITS = struct.unpack("<i", struct.pack("<f", _GAMMA))[0]
_KEY_G = (_GBITS | 0x80000000) & 0xFFFFFFFF
_HI_G = _KEY_G >> 16
_LO_G = _KEY_G & 0xFFFF


def _zero_hist(hist):
    zeros = jnp.zeros((_L,), jnp.int32)

    @plsc.parallel_loop(0, _NBINS // _L, 1, unroll=8)
    def _(i):
        hist[pl.ds(i * _L, _L)] = zeros


def _hist_body(rows, cols, lo_pass):
    rows_w = rows // _NW
    nchunk = rows_w // _CROWS
    nouter = nchunk // 2
    vec_per_row = cols // _L

    def process(buf, hist, params):
        ones = jnp.ones((_L,), jnp.int32)
        gam = jnp.float32(_GAMMA)

        def vec(i):
            for r in range(_CROWS):
                v = buf[r, pl.ds(i * _L, _L)]
                if lo_pass:
                    cv, rv = params
                    tu = lax.bitcast_convert_type(v, jnp.uint32)
                    u = tu - cv
                    m = u < rv
                    ui = lax.bitcast_convert_type(u, jnp.int32)
                    plsc.addupdate_scatter(hist, [ui], ones, mask=m)
                else:
                    m = v < gam
                    t = lax.bitcast_convert_type(v, jnp.int32)
                    hi = lax.shift_right_logical(t, 16)
                    plsc.addupdate_scatter(hist, [hi], ones, mask=m)

        plsc.parallel_loop(0, vec_per_row, 1, unroll=_UNROLL)(vec)

    def body(x_hbm, out_hbm, buf0, buf1, hist, sem0, sem1, params=None):
        wid = lax.axis_index("s") * _NC + lax.axis_index("c")
        base = wid * rows_w
        _zero_hist(hist)

        pltpu.async_copy(x_hbm.at[pl.ds(base, _CROWS)], buf0, sem0)

        def outer(h, c):
            s1 = pl.multiple_of(base + (2 * h + 1) * _CROWS, _CROWS)
            pltpu.async_copy(x_hbm.at[pl.ds(s1, _CROWS)], buf1, sem1)
            pltpu.make_async_copy(x_hbm.at[pl.ds(0, _CROWS)], buf0, sem0).wait()
            process(buf0, hist, params)

            @pl.when(h < nouter - 1)
            def _():
                s0 = pl.multiple_of(base + (2 * h + 2) * _CROWS, _CROWS)
                pltpu.async_copy(x_hbm.at[pl.ds(s0, _CROWS)], buf0, sem0)

            pltpu.make_async_copy(x_hbm.at[pl.ds(0, _CROWS)], buf1, sem1).wait()
            process(buf1, hist, params)
            return c

        lax.fori_loop(0, nouter, outer, 0)
        pltpu.sync_copy(hist, out_hbm.at[wid])

    if lo_pass:
        def body_lo(x_hbm, filt_hbm, out_hbm, buf0, buf1, filtv, hist, sem0,
                    sem1):
            pltpu.sync_copy(filt_hbm, filtv)
            cv = lax.bitcast_convert_type(filtv[0, :], jnp.uint32)
            rv = lax.bitcast_convert_type(filtv[1, :], jnp.uint32)
            body(x_hbm, out_hbm, buf0, buf1, hist, sem0, sem1,
                 params=(cv, rv))

        return body_lo

    def body_hi(x_hbm, out_hbm, buf0, buf1, hist, sem0, sem1):
        body(x_hbm, out_hbm, buf0, buf1, hist, sem0, sem1)

    return body_hi


@functools.cache
def _make_sc_kernels(rows, cols):
    mesh = plsc.VectorSubcoreMesh(core_axis_name="c", subcore_axis_name="s")
    hist_type = jax.ShapeDtypeStruct((_NW, _NBINS), jnp.int32)
    params = pltpu.CompilerParams(needs_layout_passes=False)
    hist_hi = pl.kernel(
        _hist_body(rows, cols, lo_pass=False),
        out_type=hist_type,
        mesh=mesh,
        compiler_params=params,
        scratch_types=[
            pltpu.VMEM((_CROWS, cols), jnp.float32),
            pltpu.VMEM((_CROWS, cols), jnp.float32),
            pltpu.VMEM((_NBINS,), jnp.int32),
            pltpu.SemaphoreType.DMA,
            pltpu.SemaphoreType.DMA,
        ],
    )
    hist_lo = pl.kernel(
        _hist_body(rows, cols, lo_pass=True),
        out_type=hist_type,
        mesh=mesh,
        compiler_params=params,
        scratch_types=[
            pltpu.VMEM((_CROWS, cols), jnp.float32),
            pltpu.VMEM((_CROWS, cols), jnp.float32),
            pltpu.VMEM((2, _L), jnp.int32),
            pltpu.VMEM((_NBINS,), jnp.int32),
            pltpu.SemaphoreType.DMA,
            pltpu.SemaphoreType.DMA,
        ],
    )
    return hist_hi, hist_lo


def _clamp_body(x_ref, o_ref):
    v = x_ref[...]
    o_ref[...] = jnp.where(v < _GAMMA, v, jnp.float32(_GAMMA))


def _clamp2d(x2d):
    rows, cols = x2d.shape
    br = 1024
    return pl.pallas_call(
        _clamp_body,
        grid=(rows // br,),
        in_specs=[pl.BlockSpec((br, cols), lambda i: (i, 0))],
        out_specs=pl.BlockSpec((br, cols), lambda i: (i, 0)),
        out_shape=jax.ShapeDtypeStruct((rows, cols), jnp.float32),
    )(x2d)


def kernel(x, scale):
    shp = x.shape
    total = 1
    for s in shp:
        total *= s
    cols = shp[-1]
    rows = total // cols
    x2d = x.reshape(rows, cols)

    out = _clamp2d(x2d).reshape(shp)

    hist_hi, hist_lo = _make_sc_kernels(rows, cols)
    k_rank = int(total * _P) - 1
    kp1 = jnp.int32(k_rank + 1)

    h_raw = hist_hi(x2d).sum(axis=0)
    h1 = jnp.concatenate([h_raw[_HALF:][::-1], h_raw[:_HALF]])
    c1 = jnp.cumsum(h1)
    gamma_total = jnp.int32(total) - c1[_NBINS - 1]
    bins = lax.iota(jnp.int32, _NBINS)
    c1f = c1 + jnp.where(bins >= _HI_G, gamma_total, 0)
    b = jnp.sum((c1f < kp1).astype(jnp.int32))
    h1b = h1[b] + jnp.where(b == _HI_G, gamma_total, 0)
    r = k_rank - (c1f[b] - h1b)

    pos = b >= jnp.int32(_HALF)
    bb = jnp.where(pos, b ^ jnp.int32(_HALF), jnp.int32(0xFFFF) - b)
    cbase = (bb.astype(jnp.uint32) << 16).astype(jnp.int32)
    rng = jnp.where(
        b == _HI_G,
        jnp.int32(_GBITS - (_HI_G ^ 0x8000) * 65536),
        jnp.int32(1 << 16),
    )
    filt = jnp.stack([
        jnp.full((_L,), cbase, jnp.int32),
        jnp.full((_L,), rng, jnp.int32),
    ])

    h2_raw = hist_lo(x2d, filt).sum(axis=0)
    h2 = jnp.where(pos, h2_raw, h2_raw[::-1])
    h2 = h2.at[_LO_G].add(jnp.where(b == _HI_G, gamma_total, 0))
    c2 = jnp.cumsum(h2)
    rp1 = (r + 1).astype(jnp.int32)
    lo = jnp.sum((c2 < rp1).astype(jnp.int32))

    key_u = (b.astype(jnp.uint32) << 16) | lo.astype(jnp.uint32)
    top = jnp.uint32(0x80000000)
    u = jnp.where(key_u >= top, key_u ^ top, ~key_u)
    val = lax.bitcast_convert_type(u, jnp.float32)
    new_scale = jnp.maximum(val, scale)
    return out, new_scale


# speedup vs baseline: 177.6572x; 1.0160x over previous
import functools
import struct

import jax
import jax.numpy as jnp
from jax import lax
from jax.experimental import pallas as pl
from jax.experimental.pallas import tpu as pltpu
from jax.experimental.pallas import tpu_sc as plsc

_GAMMA = 0.999
_P = 0.9995

_L = 16
_NC = 2
_NS = 16
_NW = _NC * _NS
_NBINS = 1 << 16
_HALF = 1 << 15
_CROWS = 8
_UNROLL = 2

_GBITS = struct.unpack("<i", struct.pack("<f", _GAMMA))[0]
_KEY_G = (_GBITS | 0x80000000) & 0xFFFFFFFF
_HI_G = _KEY_G >> 16
_LO_G = _KEY_G & 0xFFFF


def _zero_hist(hist):
    zeros = jnp.zeros((_L,), jnp.int32)

    @plsc.parallel_loop(0, _NBINS // _L, 1, unroll=8)
    def _(i):
        hist[pl.ds(i * _L, _L)] = zeros


def _hist_body(rows, cols, lo_pass):
    rows_w = rows // _NW
    nchunk = rows_w // _CROWS
    nouter = nchunk // 2
    vec_per_row = cols // _L

    def process(buf, hist, params):
        ones = jnp.ones((_L,), jnp.int32)
        gam = jnp.float32(_GAMMA)

        def vec(i):
            for r in range(_CROWS):
                v = buf[r, pl.ds(i * _L, _L)]
                if lo_pass:
                    cv, rv = params
                    tu = lax.bitcast_convert_type(v, jnp.uint32)
                    u = tu - cv
                    m = u < rv
                    ui = lax.bitcast_convert_type(u, jnp.int32)
                    plsc.addupdate_scatter(hist, [ui], ones, mask=m)
                else:
                    m = v < gam
                    t = lax.bitcast_convert_type(v, jnp.int32)
                    hi = lax.shift_right_logical(t, 16)
                    plsc.addupdate_scatter(hist, [hi], ones, mask=m)

        plsc.parallel_loop(0, vec_per_row, 1, unroll=_UNROLL)(vec)

    def body(x_hbm, out_hbm, buf0, buf1, hist, sem0, sem1, params=None):
        wid = lax.axis_index("s") * _NC + lax.axis_index("c")
        base = wid * rows_w
        _zero_hist(hist)

        pltpu.async_copy(x_hbm.at[pl.ds(base, _CROWS)], buf0, sem0)

        def outer(h, c):
            s1 = pl.multiple_of(base + (2 * h + 1) * _CROWS, _CROWS)
            pltpu.async_copy(x_hbm.at[pl.ds(s1, _CROWS)], buf1, sem1)
            pltpu.make_async_copy(x_hbm.at[pl.ds(0, _CROWS)], buf0, sem0).wait()
            process(buf0, hist, params)

            @pl.when(h < nouter - 1)
            def _():
                s0 = pl.multiple_of(base + (2 * h + 2) * _CROWS, _CROWS)
                pltpu.async_copy(x_hbm.at[pl.ds(s0, _CROWS)], buf0, sem0)

            pltpu.make_async_copy(x_hbm.at[pl.ds(0, _CROWS)], buf1, sem1).wait()
            process(buf1, hist, params)
            return c

        lax.fori_loop(0, nouter, outer, 0)
        pltpu.sync_copy(hist, out_hbm.at[wid])

    if lo_pass:
        def body_lo(x_hbm, filt_hbm, out_hbm, buf0, buf1, filtv, hist, sem0,
                    sem1):
            pltpu.sync_copy(filt_hbm, filtv)
            cv = lax.bitcast_convert_type(filtv[0, :], jnp.uint32)
            rv = lax.bitcast_convert_type(filtv[1, :], jnp.uint32)
            body(x_hbm, out_hbm, buf0, buf1, hist, sem0, sem1,
                 params=(cv, rv))

        return body_lo

    def body_hi(x_hbm, out_hbm, buf0, buf1, hist, sem0, sem1):
        body(x_hbm, out_hbm, buf0, buf1, hist, sem0, sem1)

    return body_hi


@functools.cache
def _make_sc_kernels(rows, cols):
    mesh = plsc.VectorSubcoreMesh(core_axis_name="c", subcore_axis_name="s")
    hist_type = jax.ShapeDtypeStruct((_NW, _NBINS), jnp.int32)
    params = pltpu.CompilerParams(needs_layout_passes=False)
    hist_hi = pl.kernel(
        _hist_body(rows, cols, lo_pass=False),
        out_type=hist_type,
        mesh=mesh,
        compiler_params=params,
        scratch_types=[
            pltpu.VMEM((_CROWS, cols), jnp.float32),
            pltpu.VMEM((_CROWS, cols), jnp.float32),
            pltpu.VMEM((_NBINS,), jnp.int32),
            pltpu.SemaphoreType.DMA,
            pltpu.SemaphoreType.DMA,
        ],
    )
    hist_lo = pl.kernel(
        _hist_body(rows, cols, lo_pass=True),
        out_type=hist_type,
        mesh=mesh,
        compiler_params=params,
        scratch_types=[
            pltpu.VMEM((_CROWS, cols), jnp.float32),
            pltpu.VMEM((_CROWS, cols), jnp.float32),
            pltpu.VMEM((2, _L), jnp.int32),
            pltpu.VMEM((_NBINS,), jnp.int32),
            pltpu.SemaphoreType.DMA,
            pltpu.SemaphoreType.DMA,
        ],
    )
    return hist_hi, hist_lo


def _clamp_body(x_ref, o_ref):
    v = x_ref[...]
    o_ref[...] = jnp.where(v < _GAMMA, v, jnp.float32(_GAMMA))


def _clamp_body2(x_ref, prev_ref, o_ref):
    del prev_ref
    v = x_ref[...]
    o_ref[...] = jnp.where(v < _GAMMA, v, jnp.float32(_GAMMA))


def _clamp_top(x2d, row_hi):
    rows, cols = x2d.shape
    br = 512
    return pl.pallas_call(
        _clamp_body,
        grid=(row_hi // br,),
        in_specs=[pl.BlockSpec((br, cols), lambda i: (i, 0))],
        out_specs=pl.BlockSpec((br, cols), lambda i: (i, 0)),
        out_shape=jax.ShapeDtypeStruct((rows, cols), jnp.float32),
    )(x2d)


def _clamp_bottom(x2d, prev, row_lo):
    rows, cols = x2d.shape
    br = 512
    base = row_lo // br
    return pl.pallas_call(
        _clamp_body2,
        grid=((rows - row_lo) // br,),
        in_specs=[
            pl.BlockSpec((br, cols), lambda i: (i + base, 0)),
            pl.BlockSpec(memory_space=pltpu.MemorySpace.HBM),
        ],
        out_specs=pl.BlockSpec((br, cols), lambda i: (i + base, 0)),
        out_shape=jax.ShapeDtypeStruct((rows, cols), jnp.float32),
        input_output_aliases={1: 0},
    )(x2d, prev)


def kernel(x, scale):
    shp = x.shape
    total = 1
    for s in shp:
        total *= s
    cols = shp[-1]
    rows = total // cols
    x2d = x.reshape(rows, cols)

    split = (rows // 2) // 512 * 512
    out_top = _clamp_top(x2d, split)

    hist_hi, hist_lo = _make_sc_kernels(rows, cols)
    k_rank = int(total * _P) - 1
    kp1 = jnp.int32(k_rank + 1)

    h_raw = hist_hi(x2d).sum(axis=0)
    h1 = jnp.concatenate([h_raw[_HALF:][::-1], h_raw[:_HALF]])
    c1 = jnp.cumsum(h1)
    gamma_total = jnp.int32(total) - c1[_NBINS - 1]
    bins = lax.iota(jnp.int32, _NBINS)
    c1f = c1 + jnp.where(bins >= _HI_G, gamma_total, 0)
    b = jnp.sum((c1f < kp1).astype(jnp.int32))
    h1b = h1[b] + jnp.where(b == _HI_G, gamma_total, 0)
    r = k_rank - (c1f[b] - h1b)

    pos = b >= jnp.int32(_HALF)
    bb = jnp.where(pos, b ^ jnp.int32(_HALF), jnp.int32(0xFFFF) - b)
    cbase = (bb.astype(jnp.uint32) << 16).astype(jnp.int32)
    rng = jnp.where(
        b == _HI_G,
        jnp.int32(_GBITS - (_HI_G ^ 0x8000) * 65536),
        jnp.int32(1 << 16),
    )
    filt = jnp.stack([
        jnp.full((_L,), cbase, jnp.int32),
        jnp.full((_L,), rng, jnp.int32),
    ])

    out_dep, filt = lax.optimization_barrier((out_top, filt))
    out = _clamp_bottom(x2d, out_dep, split).reshape(shp)

    h2_raw = hist_lo(x2d, filt).sum(axis=0)
    h2 = jnp.where(pos, h2_raw, h2_raw[::-1])
    h2 = h2.at[_LO_G].add(jnp.where(b == _HI_G, gamma_total, 0))
    c2 = jnp.cumsum(h2)
    rp1 = (r + 1).astype(jnp.int32)
    lo = jnp.sum((c2 < rp1).astype(jnp.int32))

    key_u = (b.astype(jnp.uint32) << 16) | lo.astype(jnp.uint32)
    top = jnp.uint32(0x80000000)
    u = jnp.where(key_u >= top, key_u ^ top, ~key_u)
    val = lax.bitcast_convert_type(u, jnp.float32)
    new_scale = jnp.maximum(val, scale)
    return out, new_scale
